# baseline (speedup 1.0000x reference)
import functools
import math

import jax
import jax.numpy as jnp
from jax import lax
from jax.experimental import pallas as pl
from jax.experimental.pallas import tpu as pltpu
from jax.experimental.pallas import tpu_sc as plsc

ALPHA = 0.5
TAU_C = 0.0
TAU_E = 0.0
KC = 8.0
KE = 8.0
TOPK_RATIO = 0.2
EPS = 1e-06
R_PERCENT = 0.2

_L = 16


def _iota16():
    return lax.iota(jnp.int32, 16)


def _lane(v, i):
    return jnp.sum(jnp.where(_iota16() == i, v, jnp.zeros_like(v)))


def _sigv(x):
    return 1.0 / (1.0 + jnp.exp(-x))


def _inv_s(x):
    return _lane(1.0 / jnp.full((_L,), x), 0)


def _vsqrt(x):
    xc = jnp.maximum(x, 1e-30)
    i = plsc.bitcast(xc, jnp.int32)
    i = jnp.int32(0x5F3759DF) - (i >> 1)
    y = plsc.bitcast(i, jnp.float32)
    for _ in range(4):
        y = y * (1.5 - 0.5 * xc * y * y)
    return x * y


def _sc_score(k_tok, B, H, K,
              a_hbm, c_hbm, e_hbm, aw_hbm, mf_hbm,
              w_hbm, hidx_hbm, factor_hbm,
              arow, crow, mfrow, srow, wrow, awrow, sfrow, t16f, t16i,
              sfull_sh, spos_sh, tk_sh):
    nchunk = K // _L
    kf = jnp.float32(k_tok)
    b = lax.axis_index("c")
    s = lax.axis_index("s")
    z16 = jnp.zeros((_L,), jnp.float32)

    def _rsum(ref):
        def body(i, acc):
            return acc + ref[pl.ds(i * _L, _L)]
        return jnp.sum(lax.fori_loop(0, nchunk, body, z16))

    def _stats(ref):
        def body(i, carry):
            sm, s2 = carry
            v = ref[pl.ds(i * _L, _L)]
            return sm + v, s2 + v * v
        sm, s2 = lax.fori_loop(0, nchunk, body, (z16, z16))
        rn = jnp.float32(1.0 / K)
        mu = jnp.sum(sm) * rn
        var = jnp.maximum(jnp.sum(s2) * rn - mu * mu, 0.0)
        sd = _lane(_vsqrt(jnp.full((_L,), var)), 0)
        return mu, sd

    @pl.when(s == 0)
    def _():
        pltpu.sync_copy(a_hbm.at[b], arow)
        pltpu.sync_copy(c_hbm.at[b], crow)
        pltpu.sync_copy(mf_hbm.at[b], mfrow)
        mu_a, sd_a = _stats(arow)
        mu_c, sd_c = _stats(crow)
        inv_a = _inv_s(sd_a + EPS)
        inv_c = _inv_s(sd_c + EPS)

        def sbody(i, acc):
            sl = pl.ds(i * _L, _L)
            za = (arow[sl] - mu_a) * inv_a
            zc = (crow[sl] - mu_c) * inv_c
            sv = jnp.maximum(zc, 0.0) * _sigv(za) * mfrow[sl]
            srow[sl] = sv
            return acc + sv
        ssum = jnp.sum(lax.fori_loop(0, nchunk, sbody, z16))
        inv_ssum = _inv_s(ssum + EPS)

        def wbody(i, _):
            sl = pl.ds(i * _L, _L)
            wrow[sl] = srow[sl] * inv_ssum
            return 0
        lax.fori_loop(0, nchunk, wbody, 0)
        pltpu.sync_copy(wrow, w_hbm.at[b])
        pltpu.sync_copy(srow, sfull_sh)

    @pl.when((s == 1) | (s == 2))
    def _():
        @pl.when(s == 1)
        def _():
            pltpu.sync_copy(c_hbm.at[b], arow)

        @pl.when(s == 2)
        def _():
            pltpu.sync_copy(e_hbm.at[b], arow)

        mu, sd = _stats(arow)

        def mmbody(i, carry):
            mn, mx = carry
            v = arow[pl.ds(i * _L, _L)]
            return jnp.minimum(mn, v), jnp.maximum(mx, v)
        big = jnp.full((_L,), 3.0e38, jnp.float32)
        mnv, mxv = lax.fori_loop(0, nchunk, mmbody, (big, -big))
        lo0 = jnp.min(mnv)
        hi0 = jnp.max(mxv) + 1.0

        def bis(_, carry):
            lo, hi = carry
            mid = 0.5 * (lo + hi)

            def cbody(i, acc):
                v = arow[pl.ds(i * _L, _L)]
                return acc + jnp.where(v >= mid, 1.0, 0.0)
            cnt = jnp.sum(lax.fori_loop(0, nchunk, cbody, z16))
            pred = cnt >= kf
            return (jnp.where(pred, mid, lo), jnp.where(pred, hi, mid))
        t, _hi = lax.fori_loop(0, 40, bis, (lo0, hi0))

        def gtbody(i, carry):
            sg, cg = carry
            v = arow[pl.ds(i * _L, _L)]
            m = v > t
            return sg + jnp.where(m, v, 0.0), cg + jnp.where(m, 1.0, 0.0)
        sgv, cgv = lax.fori_loop(0, nchunk, gtbody, (z16, z16))
        top_sum = jnp.sum(sgv) + (kf - jnp.sum(cgv)) * t
        tk_z = (top_sum * jnp.float32(1.0 / k_tok) - mu) * _inv_s(sd + EPS)
        t16f[...] = jnp.full((_L,), tk_z)
        pltpu.sync_copy(t16f, tk_sh.at[s - 1])

    plsc.subcore_barrier()

    @pl.when(s != 0)
    def _():
        pltpu.sync_copy(mf_hbm.at[b], mfrow)
    pltpu.sync_copy(sfull_sh, sfrow)

    spvec = z16
    for hh in range(2):
        h = s * 2 + hh
        pltpu.sync_copy(aw_hbm.at[b, h], awrow)

        def hbody(i, carry):
            nv, dv = carry
            sl = pl.ds(i * _L, _L)
            av = awrow[sl]
            return nv + av * sfrow[sl], dv + av * mfrow[sl]
        nv, dv = lax.fori_loop(0, nchunk, hbody, (z16, z16))
        sp = jnp.sum(nv) * _inv_s(jnp.sum(dv) + EPS)
        spvec = jnp.where(_iota16() == hh, sp, spvec)
    t16f[...] = spvec
    pltpu.sync_copy(t16f, spos_sh.at[s])

    plsc.subcore_barrier()

    @pl.when(s == 0)
    def _():
        io = _iota16()
        s0 = z16
        s1 = z16
        for si in range(16):
            pltpu.sync_copy(spos_sh.at[si], t16f)
            v = t16f[...]
            a0 = _lane(v, 0)
            a1 = _lane(v, 1)
            h0 = 2 * si
            if h0 < 16:
                s0 = jnp.where(io == h0, a0, s0)
            else:
                s1 = jnp.where(io == h0 - 16, a0, s1)
            h1 = 2 * si + 1
            if h1 < 16:
                s0 = jnp.where(io == h1, a1, s0)
            else:
                s1 = jnp.where(io == h1 - 16, a1, s1)

        pltpu.sync_copy(tk_sh.at[0], t16f)
        tkc = _lane(t16f[...], 0)
        pltpu.sync_copy(tk_sh.at[1], t16f)
        tke = _lane(t16f[...], 0)
        gv = _sigv(jnp.full((_L,), KC * (TAU_C - tkc))) * \
            _sigv(jnp.full((_L,), KE * (TAU_E - tke)))
        g = _lane(gv, 0)

        zi = jnp.zeros((_L,), jnp.int32)
        one = jnp.ones((_L,), jnp.int32)
        i0 = io
        i1 = io + 16
        rank0 = zi
        rank1 = zi
        for j in range(H):
            sj = _lane(s0, j) if j < 16 else _lane(s1, j - 16)
            rank0 = rank0 + jnp.where(
                (sj > s0) | ((sj == s0) & (j < i0)), one, zi)
            rank1 = rank1 + jnp.where(
                (sj > s1) | ((sj == s1) & (j < i1)), one, zi)

        hvec = zi
        fvec = z16
        for j in range(7):
            e0 = rank0 == j
            e1 = rank1 == j
            hj = jnp.sum(jnp.where(e0, i0, zi)) + \
                jnp.sum(jnp.where(e1, i1, zi))
            spj = jnp.sum(jnp.where(e0, s0, z16)) + \
                jnp.sum(jnp.where(e1, s1, z16))
            fj = jnp.float32(ALPHA) * g * jnp.where(spj > 0.0, 1.0, 0.0)
            hvec = jnp.where(io == j, hj, hvec)
            fvec = jnp.where(io == j, fj, fvec)
        t16i[...] = hvec
        pltpu.sync_copy(t16i, hidx_hbm.at[b])
        t16f[...] = fvec
        pltpu.sync_copy(t16f, factor_hbm.at[b])


def kernel(attn_output, value_states, A, C, E, D, attn_weights_last,
           image_mask):
    del D
    B, H, Q, DH = attn_output.shape
    K = value_states.shape[2]
    k_tok = min(max(1, int(math.ceil(TOPK_RATIO * K))), K)
    k_heads = min(max(1, int(math.ceil(R_PERCENT * H))), H)

    mf = image_mask.astype(jnp.float32)

    mesh = plsc.VectorSubcoreMesh(core_axis_name="c", subcore_axis_name="s")
    w, hidx, factor = pl.kernel(
        functools.partial(_sc_score, k_tok, B, H, K),
        out_type=[
            jax.ShapeDtypeStruct((B, K), jnp.float32),
            jax.ShapeDtypeStruct((B, _L), jnp.int32),
            jax.ShapeDtypeStruct((B, _L), jnp.float32),
        ],
        mesh=mesh,
        compiler_params=pltpu.CompilerParams(needs_layout_passes=False),
        scratch_types=[
            pltpu.VMEM((K,), jnp.float32),
            pltpu.VMEM((K,), jnp.float32),
            pltpu.VMEM((K,), jnp.float32),
            pltpu.VMEM((K,), jnp.float32),
            pltpu.VMEM((K,), jnp.float32),
            pltpu.VMEM((K,), jnp.float32),
            pltpu.VMEM((K,), jnp.float32),
            pltpu.VMEM((_L,), jnp.float32),
            pltpu.VMEM((_L,), jnp.int32),
            pltpu.VMEM_SHARED((K,), jnp.float32),
            pltpu.VMEM_SHARED((16, _L), jnp.float32),
            pltpu.VMEM_SHARED((2, _L), jnp.float32),
        ],
    )(A, C, E, attn_weights_last, mf)

    def _upd(hidx_sm, factor_sm, v_ref, w_ref, attn_ref, out_ref):
        b = pl.program_id(0)
        j = pl.program_id(1)

        @pl.when(j == 0)
        def _():
            out_ref[...] = attn_ref[...]

        wv = jax.lax.dot_general(
            w_ref[0], v_ref[0, 0], (((1,), (0,)), ((), ())),
            preferred_element_type=jnp.float32)
        f = factor_sm[b, j]
        h = hidx_sm[b, j]
        out_ref[0, pl.ds(h, 1), 7, :] += f * wv

    grid_spec = pltpu.PrefetchScalarGridSpec(
        num_scalar_prefetch=2,
        grid=(B, k_heads),
        in_specs=[
            pl.BlockSpec((1, 1, K, DH),
                         lambda b, j, hidx, factor: (b, hidx[b, j], 0, 0)),
            pl.BlockSpec((1, 1, K), lambda b, j, hidx, factor: (b, 0, 0)),
            pl.BlockSpec((1, H, 8, DH),
                         lambda b, j, hidx, factor: (b, 0, Q // 8 - 1, 0)),
        ],
        out_specs=pl.BlockSpec((1, H, 8, DH),
                               lambda b, j, hidx, factor: (b, 0, Q // 8 - 1, 0)),
    )

    out = pl.pallas_call(
        _upd,
        grid_spec=grid_spec,
        out_shape=jax.ShapeDtypeStruct((B, H, Q, DH), jnp.float32),
        input_output_aliases={4: 0},
    )(hidx, factor, value_states, w.reshape(B, 1, K), attn_output)
    return out


# speedup vs baseline: 1.2685x; 1.2685x over previous
import functools
import math

import jax
import jax.numpy as jnp
from jax import lax
from jax.experimental import pallas as pl
from jax.experimental.pallas import tpu as pltpu
from jax.experimental.pallas import tpu_sc as plsc

ALPHA = 0.5
TAU_C = 0.0
TAU_E = 0.0
KC = 8.0
KE = 8.0
TOPK_RATIO = 0.2
EPS = 1e-06
R_PERCENT = 0.2

_L = 16


def _iota16():
    return lax.iota(jnp.int32, 16)


def _lane(v, i):
    return jnp.sum(jnp.where(_iota16() == i, v, jnp.zeros_like(v)))


def _sigv(x):
    return 1.0 / (1.0 + jnp.exp(-x))


def _inv_s(x):
    return _lane(1.0 / jnp.full((_L,), x), 0)


def _vsqrt(x):
    xc = jnp.maximum(x, 1e-30)
    i = plsc.bitcast(xc, jnp.int32)
    i = jnp.int32(0x5F3759DF) - (i >> 1)
    y = plsc.bitcast(i, jnp.float32)
    for _ in range(4):
        y = y * (1.5 - 0.5 * xc * y * y)
    return x * y


def _sc_score(k_tok, B, H, K,
              a_hbm, c_hbm, e_hbm, aw_hbm, mf_hbm,
              w_hbm, hidx_hbm, factor_hbm,
              arow, crow, mfrow, srow, wrow, awrow, sfrow, t16f, t16i,
              sfull_sh, spos_sh, tk_sh):
    nchunk = K // _L
    kf = jnp.float32(k_tok)
    b = lax.axis_index("c")
    s = lax.axis_index("s")
    z16 = jnp.zeros((_L,), jnp.float32)

    def _rsum(ref):
        def body(i, acc):
            return acc + ref[pl.ds(i * _L, _L)]
        return jnp.sum(lax.fori_loop(0, nchunk, body, z16))

    def _stats(ref):
        def body(i, carry):
            sm, s2 = carry
            v = ref[pl.ds(i * _L, _L)]
            return sm + v, s2 + v * v
        sm, s2 = lax.fori_loop(0, nchunk, body, (z16, z16))
        rn = jnp.float32(1.0 / K)
        mu = jnp.sum(sm) * rn
        var = jnp.maximum(jnp.sum(s2) * rn - mu * mu, 0.0)
        sd = _lane(_vsqrt(jnp.full((_L,), var)), 0)
        return mu, sd

    @pl.when(s == 0)
    def _():
        pltpu.sync_copy(a_hbm.at[b], arow)
        pltpu.sync_copy(c_hbm.at[b], crow)
        pltpu.sync_copy(mf_hbm.at[b], mfrow)
        mu_a, sd_a = _stats(arow)
        mu_c, sd_c = _stats(crow)
        inv_a = _inv_s(sd_a + EPS)
        inv_c = _inv_s(sd_c + EPS)

        def sbody(i, acc):
            sl = pl.ds(i * _L, _L)
            za = (arow[sl] - mu_a) * inv_a
            zc = (crow[sl] - mu_c) * inv_c
            sv = jnp.maximum(zc, 0.0) * _sigv(za) * mfrow[sl]
            srow[sl] = sv
            return acc + sv
        ssum = jnp.sum(lax.fori_loop(0, nchunk, sbody, z16))
        inv_ssum = _inv_s(ssum + EPS)

        def wbody(i, _):
            sl = pl.ds(i * _L, _L)
            wrow[sl] = srow[sl] * inv_ssum
            return 0
        lax.fori_loop(0, nchunk, wbody, 0)
        pltpu.sync_copy(wrow, w_hbm.at[b])
        pltpu.sync_copy(srow, sfull_sh)

    @pl.when((s == 1) | (s == 2))
    def _():
        @pl.when(s == 1)
        def _():
            pltpu.sync_copy(c_hbm.at[b], arow)

        @pl.when(s == 2)
        def _():
            pltpu.sync_copy(e_hbm.at[b], arow)

        mu, sd = _stats(arow)

        def mmbody(i, carry):
            mn, mx = carry
            v = arow[pl.ds(i * _L, _L)]
            return jnp.minimum(mn, v), jnp.maximum(mx, v)
        big = jnp.full((_L,), 3.0e38, jnp.float32)
        mnv, mxv = lax.fori_loop(0, nchunk, mmbody, (big, -big))
        lo0 = jnp.min(mnv)
        hi0 = jnp.max(mxv) + 1.0

        def bis(_, carry):
            lo, hi = carry
            mid = 0.5 * (lo + hi)

            def cbody(i, acc):
                v = arow[pl.ds(i * _L, _L)]
                return acc + jnp.where(v >= mid, 1.0, 0.0)
            cnt = jnp.sum(lax.fori_loop(0, nchunk, cbody, z16))
            pred = cnt >= kf
            return (jnp.where(pred, mid, lo), jnp.where(pred, hi, mid))
        t, _hi = lax.fori_loop(0, 30, bis, (lo0, hi0))

        def gtbody(i, carry):
            sg, cg = carry
            v = arow[pl.ds(i * _L, _L)]
            m = v > t
            return sg + jnp.where(m, v, 0.0), cg + jnp.where(m, 1.0, 0.0)
        sgv, cgv = lax.fori_loop(0, nchunk, gtbody, (z16, z16))
        top_sum = jnp.sum(sgv) + (kf - jnp.sum(cgv)) * t
        tk_z = (top_sum * jnp.float32(1.0 / k_tok) - mu) * _inv_s(sd + EPS)
        t16f[...] = jnp.full((_L,), tk_z)
        pltpu.sync_copy(t16f, tk_sh.at[s - 1])

    plsc.subcore_barrier()

    @pl.when(s != 0)
    def _():
        pltpu.sync_copy(mf_hbm.at[b], mfrow)
    pltpu.sync_copy(sfull_sh, sfrow)

    spvec = z16
    for hh in range(2):
        h = s * 2 + hh
        pltpu.sync_copy(aw_hbm.at[b, h], awrow)

        def hbody(i, carry):
            nv, dv = carry
            sl = pl.ds(i * _L, _L)
            av = awrow[sl]
            return nv + av * sfrow[sl], dv + av * mfrow[sl]
        nv, dv = lax.fori_loop(0, nchunk, hbody, (z16, z16))
        sp = jnp.sum(nv) * _inv_s(jnp.sum(dv) + EPS)
        spvec = jnp.where(_iota16() == hh, sp, spvec)
    t16f[...] = spvec
    pltpu.sync_copy(t16f, spos_sh.at[s])

    plsc.subcore_barrier()

    @pl.when(s == 0)
    def _():
        io = _iota16()
        s0 = z16
        s1 = z16
        for si in range(16):
            pltpu.sync_copy(spos_sh.at[si], t16f)
            v = t16f[...]
            a0 = _lane(v, 0)
            a1 = _lane(v, 1)
            h0 = 2 * si
            if h0 < 16:
                s0 = jnp.where(io == h0, a0, s0)
            else:
                s1 = jnp.where(io == h0 - 16, a0, s1)
            h1 = 2 * si + 1
            if h1 < 16:
                s0 = jnp.where(io == h1, a1, s0)
            else:
                s1 = jnp.where(io == h1 - 16, a1, s1)

        pltpu.sync_copy(tk_sh.at[0], t16f)
        tkc = _lane(t16f[...], 0)
        pltpu.sync_copy(tk_sh.at[1], t16f)
        tke = _lane(t16f[...], 0)
        gv = _sigv(jnp.full((_L,), KC * (TAU_C - tkc))) * \
            _sigv(jnp.full((_L,), KE * (TAU_E - tke)))
        g = _lane(gv, 0)

        zi = jnp.zeros((_L,), jnp.int32)
        one = jnp.ones((_L,), jnp.int32)
        i0 = io
        i1 = io + 16
        rank0 = zi
        rank1 = zi
        for j in range(H):
            sj = _lane(s0, j) if j < 16 else _lane(s1, j - 16)
            rank0 = rank0 + jnp.where(
                (sj > s0) | ((sj == s0) & (j < i0)), one, zi)
            rank1 = rank1 + jnp.where(
                (sj > s1) | ((sj == s1) & (j < i1)), one, zi)

        hvec = zi
        fvec = z16
        for j in range(7):
            e0 = rank0 == j
            e1 = rank1 == j
            hj = jnp.sum(jnp.where(e0, i0, zi)) + \
                jnp.sum(jnp.where(e1, i1, zi))
            spj = jnp.sum(jnp.where(e0, s0, z16)) + \
                jnp.sum(jnp.where(e1, s1, z16))
            fj = jnp.float32(ALPHA) * g * jnp.where(spj > 0.0, 1.0, 0.0)
            hvec = jnp.where(io == j, hj, hvec)
            fvec = jnp.where(io == j, fj, fvec)
        t16i[...] = hvec
        pltpu.sync_copy(t16i, hidx_hbm.at[b])
        t16f[...] = fvec
        pltpu.sync_copy(t16f, factor_hbm.at[b])


def kernel(attn_output, value_states, A, C, E, D, attn_weights_last,
           image_mask):
    del D
    B, H, Q, DH = attn_output.shape
    K = value_states.shape[2]
    k_tok = min(max(1, int(math.ceil(TOPK_RATIO * K))), K)
    k_heads = min(max(1, int(math.ceil(R_PERCENT * H))), H)

    mf = image_mask.astype(jnp.float32)

    mesh = plsc.VectorSubcoreMesh(core_axis_name="c", subcore_axis_name="s")
    w, hidx, factor = pl.kernel(
        functools.partial(_sc_score, k_tok, B, H, K),
        out_type=[
            jax.ShapeDtypeStruct((B, K), jnp.float32),
            jax.ShapeDtypeStruct((B, _L), jnp.int32),
            jax.ShapeDtypeStruct((B, _L), jnp.float32),
        ],
        mesh=mesh,
        compiler_params=pltpu.CompilerParams(needs_layout_passes=False),
        scratch_types=[
            pltpu.VMEM((K,), jnp.float32),
            pltpu.VMEM((K,), jnp.float32),
            pltpu.VMEM((K,), jnp.float32),
            pltpu.VMEM((K,), jnp.float32),
            pltpu.VMEM((K,), jnp.float32),
            pltpu.VMEM((K,), jnp.float32),
            pltpu.VMEM((K,), jnp.float32),
            pltpu.VMEM((_L,), jnp.float32),
            pltpu.VMEM((_L,), jnp.int32),
            pltpu.VMEM_SHARED((K,), jnp.float32),
            pltpu.VMEM_SHARED((16, _L), jnp.float32),
            pltpu.VMEM_SHARED((2, _L), jnp.float32),
        ],
    )(A, C, E, attn_weights_last, mf)

    def _copy(src_ref, dst_ref):
        dst_ref[...] = src_ref[...]

    rows = B * H * Q
    blk = 4096
    flat = attn_output.reshape(rows, DH)
    attn_copy = pl.pallas_call(
        _copy,
        grid=(rows // blk,),
        in_specs=[pl.BlockSpec((blk, DH), lambda i: (i, 0))],
        out_specs=pl.BlockSpec((blk, DH), lambda i: (i, 0)),
        out_shape=jax.ShapeDtypeStruct((rows, DH), jnp.float32),
    )(flat).reshape(B, H, Q, DH)

    def _upd(hidx_sm, factor_sm, v_ref, w_ref, attn_ref, out_ref):
        b = pl.program_id(0)
        j = pl.program_id(1)

        @pl.when(j == 0)
        def _():
            out_ref[...] = attn_ref[...]

        wv = jax.lax.dot_general(
            w_ref[0], v_ref[0, 0], (((1,), (0,)), ((), ())),
            preferred_element_type=jnp.float32)
        f = factor_sm[b, j]
        h = hidx_sm[b, j]
        out_ref[0, pl.ds(h, 1), 7, :] += f * wv

    grid_spec = pltpu.PrefetchScalarGridSpec(
        num_scalar_prefetch=2,
        grid=(B, k_heads),
        in_specs=[
            pl.BlockSpec((1, 1, K, DH),
                         lambda b, j, hidx, factor: (b, hidx[b, j], 0, 0)),
            pl.BlockSpec((1, 1, K), lambda b, j, hidx, factor: (b, 0, 0)),
            pl.BlockSpec((1, H, 8, DH),
                         lambda b, j, hidx, factor: (b, 0, Q // 8 - 1, 0)),
        ],
        out_specs=pl.BlockSpec((1, H, 8, DH),
                               lambda b, j, hidx, factor: (b, 0, Q // 8 - 1, 0)),
    )

    out = pl.pallas_call(
        _upd,
        grid_spec=grid_spec,
        out_shape=jax.ShapeDtypeStruct((B, H, Q, DH), jnp.float32),
        input_output_aliases={4: 0},
    )(hidx, factor, value_states, w.reshape(B, 1, K), attn_copy)
    return out


# speedup vs baseline: 1.4496x; 1.1427x over previous
import functools
import math

import jax
import jax.numpy as jnp
from jax.experimental import pallas as pl
from jax.experimental.pallas import tpu as pltpu

ALPHA = 0.5
TAU_C = 0.0
TAU_E = 0.0
KC = 8.0
KE = 8.0
TOPK_RATIO = 0.2
EPS = 1e-06
R_PERCENT = 0.2


def _sig(x):
    return 1.0 / (1.0 + jnp.exp(-x))


def _zscore(x):
    mu = jnp.mean(x, axis=-1, keepdims=True)
    var = jnp.mean((x - mu) ** 2, axis=-1, keepdims=True)
    return (x - mu) / (jnp.sqrt(var) + EPS)


def _score_kernel(k_tok, k_heads, a_ref, c_ref, e_ref, aw_ref, mf_ref,
                  w_ref, hidx_ref, factor_ref):
    B, K = a_ref.shape
    H = aw_ref.shape[1]
    za = _zscore(a_ref[...])
    zc = _zscore(c_ref[...])
    ze = _zscore(e_ref[...])
    mf = mf_ref[...]

    s_full = jnp.maximum(zc, 0.0) * _sig(za) * mf
    ssum = jnp.sum(s_full, axis=-1, keepdims=True)
    w = s_full / (ssum + EPS)
    w_ref[...] = w

    x4 = jnp.concatenate([zc, ze], axis=0)
    lo0 = jnp.min(x4, axis=-1, keepdims=True)
    hi0 = jnp.max(x4, axis=-1, keepdims=True) + 1.0
    kf = jnp.float32(k_tok)

    def _bisect(_, carry):
        lo, hi = carry
        mid = (lo + hi) * 0.5
        cnt = jnp.sum((x4 >= mid).astype(jnp.float32), axis=-1, keepdims=True)
        pred = cnt >= kf
        return jnp.where(pred, mid, lo), jnp.where(pred, hi, mid)

    lo, _ = jax.lax.fori_loop(0, 48, _bisect, (lo0, hi0))
    gt = (x4 > lo).astype(jnp.float32)
    cnt_gt = jnp.sum(gt, axis=-1, keepdims=True)
    top_sum = jnp.sum(x4 * gt, axis=-1, keepdims=True) + (kf - cnt_gt) * lo
    tk = top_sum / kf
    tkc, tke = tk[:B], tk[B:]
    g = _sig(KC * (TAU_C - tkc)) * _sig(KE * (TAU_E - tke))

    aw = aw_ref[...]
    num = jnp.sum(aw * s_full[:, None, :], axis=-1)
    den = jnp.sum(aw * mf[:, None, :], axis=-1)
    s_pos = num / (den + EPS)

    sj = s_pos[:, :, None]
    sh = s_pos[:, None, :]
    jj = jax.lax.broadcasted_iota(jnp.int32, (B, H, H), 1)
    hh = jax.lax.broadcasted_iota(jnp.int32, (B, H, H), 2)
    beats = (sj > sh) | ((sj == sh) & (jj < hh))
    rank = jnp.sum(beats.astype(jnp.int32), axis=1)

    h_iota = jax.lax.broadcasted_iota(jnp.int32, (B, H), 1)
    hsel, fsel = [], []
    for j in range(k_heads):
        eqj = rank == j
        hsel.append(jnp.sum(jnp.where(eqj, h_iota, 0), axis=1, keepdims=True))
        spj = jnp.sum(jnp.where(eqj, s_pos, 0.0), axis=1, keepdims=True)
        fsel.append((spj > 0.0).astype(jnp.float32))
    hidx_ref[...] = jnp.concatenate(hsel, axis=1)
    factor_ref[...] = jnp.float32(ALPHA) * g * jnp.concatenate(fsel, axis=1)


def kernel(attn_output, value_states, A, C, E, D, attn_weights_last,
           image_mask):
    del D
    B, H, Q, DH = attn_output.shape
    K = value_states.shape[2]
    k_tok = min(max(1, int(math.ceil(TOPK_RATIO * K))), K)
    k_heads = min(max(1, int(math.ceil(R_PERCENT * H))), H)

    mf = image_mask.astype(jnp.float32)

    w, hidx, factor = pl.pallas_call(
        functools.partial(_score_kernel, k_tok, k_heads),
        grid=(1,),
        in_specs=[
            pl.BlockSpec((B, K), lambda i: (0, 0)),
            pl.BlockSpec((B, K), lambda i: (0, 0)),
            pl.BlockSpec((B, K), lambda i: (0, 0)),
            pl.BlockSpec((B, H, K), lambda i: (0, 0, 0)),
            pl.BlockSpec((B, K), lambda i: (0, 0)),
        ],
        out_specs=[
            pl.BlockSpec((B, K), lambda i: (0, 0)),
            pl.BlockSpec((B, k_heads), lambda i: (0, 0)),
            pl.BlockSpec((B, k_heads), lambda i: (0, 0)),
        ],
        out_shape=[
            jax.ShapeDtypeStruct((B, K), jnp.float32),
            jax.ShapeDtypeStruct((B, k_heads), jnp.int32),
            jax.ShapeDtypeStruct((B, k_heads), jnp.float32),
        ],
    )(A, C, E, attn_weights_last, mf)

    def _copy(src_ref, dst_ref):
        dst_ref[...] = src_ref[...]

    rows = B * H * Q
    blk = 4096
    flat = attn_output.reshape(rows, DH)
    attn_copy = pl.pallas_call(
        _copy,
        grid=(rows // blk,),
        in_specs=[pl.BlockSpec((blk, DH), lambda i: (i, 0))],
        out_specs=pl.BlockSpec((blk, DH), lambda i: (i, 0)),
        out_shape=jax.ShapeDtypeStruct((rows, DH), jnp.float32),
    )(flat).reshape(B, H, Q, DH)

    def _upd(hidx_sm, factor_sm, v_ref, w_ref, attn_ref, out_ref):
        b = pl.program_id(0)
        j = pl.program_id(1)

        @pl.when(j == 0)
        def _():
            out_ref[...] = attn_ref[...]

        wv = jax.lax.dot_general(
            w_ref[0], v_ref[0, 0], (((1,), (0,)), ((), ())),
            preferred_element_type=jnp.float32)
        f = factor_sm[b, j]
        h = hidx_sm[b, j]
        out_ref[0, pl.ds(h, 1), 7, :] += f * wv

    grid_spec = pltpu.PrefetchScalarGridSpec(
        num_scalar_prefetch=2,
        grid=(B, k_heads),
        in_specs=[
            pl.BlockSpec((1, 1, K, DH),
                         lambda b, j, hidx, factor: (b, hidx[b, j], 0, 0)),
            pl.BlockSpec((1, 1, K), lambda b, j, hidx, factor: (b, 0, 0)),
            pl.BlockSpec((1, H, 8, DH),
                         lambda b, j, hidx, factor: (b, 0, Q // 8 - 1, 0)),
        ],
        out_specs=pl.BlockSpec((1, H, 8, DH),
                               lambda b, j, hidx, factor: (b, 0, Q // 8 - 1, 0)),
    )

    out = pl.pallas_call(
        _upd,
        grid_spec=grid_spec,
        out_shape=jax.ShapeDtypeStruct((B, H, Q, DH), jnp.float32),
        input_output_aliases={4: 0},
    )(hidx, factor, value_states, w.reshape(B, 1, K), attn_copy)
    return out
